# baseline (speedup 1.0000x reference)
import functools
import math

import jax
import jax.numpy as jnp
from jax import lax
from jax.experimental import pallas as pl
from jax.experimental.pallas import tpu as pltpu
from jax.experimental.pallas import tpu_sc as plsc

B, S, HIDDEN = 1, 2048, 1024
FFN = 2048
E = 8
K = 2
T = B * S
CAP = int(math.ceil(T * K / E * 1.25))
CAPP = 656
DROP_POS = CAPP - 1
ROWS = E * CAPP
NP = 2 * T

_SC_MESH = plsc.VectorSubcoreMesh(
    core_axis_name="c", subcore_axis_name="s", num_cores=2, num_subcores=16)
_NW = 32


def _wid():
    return lax.axis_index("s") * 2 + lax.axis_index("c")


def _router_body(x_ref, rw_ref, rb_ref, eid_ref, wgt_ref):
    xf = x_ref[...]
    logits = lax.dot_general(xf, rw_ref[...], (((1,), (1,)), ((), ())),
                             preferred_element_type=jnp.float32)
    logits = logits + rb_ref[...]
    probs = jax.nn.softmax(logits, axis=-1)
    ei = lax.broadcasted_iota(jnp.int32, (T, E), 1)
    m1 = jnp.max(probs, axis=1, keepdims=True)
    i1 = jnp.min(jnp.where(probs == m1, ei, E), axis=1, keepdims=True)
    pm = jnp.where(ei == i1, -1.0, probs)
    m2 = jnp.max(pm, axis=1, keepdims=True)
    i2 = jnp.min(jnp.where(pm == m2, ei, E), axis=1, keepdims=True)
    s = m1 + m2
    eid_ref[...] = jnp.concatenate([i1, i2], axis=1)
    wgt_ref[...] = jnp.concatenate([m1 / s, m2 / s], axis=1)


def _router(xf, router_w, router_b):
    return pl.pallas_call(
        _router_body,
        out_shape=(jax.ShapeDtypeStruct((T, K), jnp.int32),
                   jax.ShapeDtypeStruct((T, K), jnp.float32)),
    )(xf, router_w, router_b.reshape(1, E))


@functools.partial(
    pl.kernel,
    out_type=(jax.ShapeDtypeStruct((ROWS,), jnp.int32),
              jax.ShapeDtypeStruct((ROWS,), jnp.float32),
              jax.ShapeDtypeStruct((NP,), jnp.int32)),
    mesh=_SC_MESH,
    scratch_types=[pltpu.VMEM((NP,), jnp.int32),
                   pltpu.VMEM((NP,), jnp.float32),
                   pltpu.VMEM((ROWS,), jnp.int32),
                   pltpu.VMEM((ROWS,), jnp.float32),
                   pltpu.VMEM((NP,), jnp.int32),
                   pltpu.VMEM((16,), jnp.int32)],
    compiler_params=pltpu.CompilerParams(needs_layout_passes=False),
)
def _route(eid_hbm, wgt_hbm, src_hbm, wslot_hbm, slot_hbm,
           eid_v, wgt_v, src_v, wslot_v, slot_v, cnt_v):
    @pl.when(_wid() == 0)
    def _():
        pltpu.sync_copy(eid_hbm, eid_v)
        pltpu.sync_copy(wgt_hbm, wgt_v)
        zi = jnp.zeros((16,), jnp.int32)
        zf = jnp.zeros((16,), jnp.float32)
        cnt_v[...] = zi

        def init_body(i, c):
            src_v[pl.ds(i * 16, 16)] = zi
            wslot_v[pl.ds(i * 16, 16)] = zf
            return c
        lax.fori_loop(0, ROWS // 16, init_body, 0)

        lane = lax.iota(jnp.int32, 16)

        def body(i, c):
            ids = eid_v[pl.ds(i * 16, 16)]
            wv = wgt_v[pl.ds(i * 16, 16)]
            base = plsc.load_gather(cnt_v, [ids])
            prefix = zi
            tot = zi
            for e in range(E):
                sel = ids == e
                ind = jnp.where(sel, 1, 0)
                cs = plsc.cumsum(ind)
                prefix = jnp.where(sel, cs - 1, prefix)
                tot = jnp.where(sel, jnp.max(cs), tot)
            pos = base + prefix
            plsc.store_scatter(cnt_v, [ids], base + tot)
            keep = pos < CAP
            slot = ids * CAPP + jnp.where(keep, pos, DROP_POS)
            slot_v[pl.ds(i * 16, 16)] = slot
            tok = lax.shift_right_logical(i * 16 + lane, 1)
            plsc.store_scatter(src_v, [slot], tok, mask=keep)
            plsc.store_scatter(wslot_v, [slot], wv, mask=keep)
            return c
        lax.fori_loop(0, NP // 16, body, 0)
        pltpu.sync_copy(src_v, src_hbm)
        pltpu.sync_copy(wslot_v, wslot_hbm)
        pltpu.sync_copy(slot_v, slot_hbm)


_GCH = 64
_NGCH = ROWS // _GCH
_GPW = -(-_NGCH // _NW)
_HW2 = HIDDEN // 2


@functools.partial(
    pl.kernel,
    out_type=jax.ShapeDtypeStruct((ROWS, _HW2), jnp.int32),
    mesh=_SC_MESH,
    scratch_types=[pltpu.VMEM((_GPW, _GCH), jnp.int32),
                   pltpu.VMEM((2, _GCH, _HW2), jnp.int32),
                   pltpu.SemaphoreType.DMA,
                   pltpu.SemaphoreType.DMA,
                   pltpu.SemaphoreType.DMA,
                   pltpu.SemaphoreType.DMA],
)
def _gather_disp(x_hbm, src_hbm, disp_hbm, idx_v, rows_v, gsem0, gsem1, wsem0, wsem1):
    w = _wid()
    gsem = (gsem0, gsem1)
    wsem = (wsem0, wsem1)

    def valid(c):
        return w + _NW * c < _NGCH

    def base(c):
        return (w + _NW * c) * _GCH

    for c in range(_GPW):
        @pl.when(valid(c))
        def _(c=c):
            pltpu.sync_copy(src_hbm.at[pl.ds(base(c), _GCH)], idx_v.at[c])

    def start_gather(c):
        pltpu.async_copy(x_hbm.at[idx_v.at[c]], rows_v.at[c % 2], gsem[c % 2])

    def wait_gather(c):
        pltpu.make_async_copy(x_hbm.at[idx_v.at[c]], rows_v.at[c % 2],
                              gsem[c % 2]).wait()

    def start_write(c):
        pltpu.async_copy(rows_v.at[c % 2], disp_hbm.at[pl.ds(base(c), _GCH)],
                         wsem[c % 2])

    def wait_write(c):
        pltpu.make_async_copy(rows_v.at[c % 2],
                              disp_hbm.at[pl.ds(base(c), _GCH)],
                              wsem[c % 2]).wait()

    @pl.when(valid(0))
    def _():
        start_gather(0)
    for c in range(_GPW):
        if c + 1 < _GPW:
            @pl.when(valid(c + 1))
            def _(c=c):
                if c - 1 >= 0:
                    wait_write(c - 1)
                start_gather(c + 1)

        @pl.when(valid(c))
        def _(c=c):
            wait_gather(c)
            start_write(c)
    for k in range(_GPW):
        if k + 2 < _GPW:
            cond = jnp.logical_and(valid(k), jnp.logical_not(valid(k + 2)))
        else:
            cond = valid(k)

        @pl.when(cond)
        def _(k=k):
            wait_write(k)


_FBLK = 512
_NF = FFN // _FBLK


def _ffn_body(disp_ref, gw_ref, uw_ref, dw_ref, w_ref, out_ref):
    f = pl.program_id(1)
    d = disp_ref[...]
    g = lax.dot_general(d, gw_ref[0].astype(jnp.bfloat16),
                        (((1,), (1,)), ((), ())),
                        preferred_element_type=jnp.float32)
    u = lax.dot_general(d, uw_ref[0].astype(jnp.bfloat16),
                        (((1,), (1,)), ((), ())),
                        preferred_element_type=jnp.float32)
    h = ((g * jax.nn.sigmoid(g)) * u).astype(jnp.bfloat16)
    p = lax.dot_general(h, dw_ref[0].astype(jnp.bfloat16),
                        (((1,), (1,)), ((), ())),
                        preferred_element_type=jnp.float32)

    @pl.when(f == 0)
    def _():
        out_ref[...] = p

    @pl.when(f != 0)
    def _():
        out_ref[...] = out_ref[...] + p

    @pl.when(f == _NF - 1)
    def _():
        out_ref[...] = out_ref[...] * w_ref[...]


def _ffn(disp, gate_w, up_w, down_w, wcol):
    return pl.pallas_call(
        _ffn_body,
        grid=(E, _NF),
        in_specs=[
            pl.BlockSpec((CAPP, HIDDEN), lambda e, f: (e, 0)),
            pl.BlockSpec((1, _FBLK, HIDDEN), lambda e, f: (e, f, 0)),
            pl.BlockSpec((1, _FBLK, HIDDEN), lambda e, f: (e, f, 0)),
            pl.BlockSpec((1, HIDDEN, _FBLK), lambda e, f: (e, 0, f)),
            pl.BlockSpec((CAPP, 1), lambda e, f: (e, 0)),
        ],
        out_specs=pl.BlockSpec((CAPP, HIDDEN), lambda e, f: (e, 0)),
        out_shape=jax.ShapeDtypeStruct((ROWS, HIDDEN), jnp.float32),
        compiler_params=pltpu.CompilerParams(
            dimension_semantics=("parallel", "arbitrary")),
    )(disp, gate_w, up_w, down_w, wcol)


_TCH = 16
_CPW = T // _TCH // _NW


@functools.partial(
    pl.kernel,
    out_type=jax.ShapeDtypeStruct((T, HIDDEN), jnp.float32),
    mesh=_SC_MESH,
    scratch_types=[pltpu.VMEM((_CPW, 2 * _TCH), jnp.int32),
                   pltpu.VMEM((2, 2 * _TCH, HIDDEN), jnp.float32),
                   pltpu.VMEM((2, _TCH, HIDDEN), jnp.float32),
                   pltpu.SemaphoreType.DMA,
                   pltpu.SemaphoreType.DMA,
                   pltpu.SemaphoreType.DMA,
                   pltpu.SemaphoreType.DMA],
)
def _combine(outbuf_hbm, slot_hbm, out_hbm, idx_v, rows_v, out_v,
             gsem0, gsem1, wsem0, wsem1):
    w = _wid()
    gsem = (gsem0, gsem1)
    wsem = (wsem0, wsem1)

    def tb(c):
        return (w * _CPW + c) * _TCH

    for c in range(_CPW):
        pltpu.sync_copy(slot_hbm.at[pl.ds(tb(c) * 2, 2 * _TCH)], idx_v.at[c])

    def start_gather(c):
        pltpu.async_copy(outbuf_hbm.at[idx_v.at[c]], rows_v.at[c % 2],
                         gsem[c % 2])

    def wait_gather(c):
        pltpu.make_async_copy(outbuf_hbm.at[idx_v.at[c]], rows_v.at[c % 2],
                              gsem[c % 2]).wait()

    def start_write(c):
        pltpu.async_copy(out_v.at[c % 2], out_hbm.at[pl.ds(tb(c), _TCH)],
                         wsem[c % 2])

    def wait_write(c):
        pltpu.make_async_copy(out_v.at[c % 2], out_hbm.at[pl.ds(tb(c), _TCH)],
                              wsem[c % 2]).wait()

    start_gather(0)
    for c in range(_CPW):
        if c + 1 < _CPW:
            if c - 1 >= 0:
                wait_write(c - 1)
            start_gather(c + 1)
        wait_gather(c)

        def tbody(t, carry, c=c):
            for v in range(HIDDEN // 16):
                col = v * 16
                out_v[c % 2, t, pl.ds(col, 16)] = (
                    rows_v[c % 2, 2 * t, pl.ds(col, 16)]
                    + rows_v[c % 2, 2 * t + 1, pl.ds(col, 16)])
            return carry
        lax.fori_loop(0, _TCH, tbody, 0)
        start_write(c)
    wait_write(_CPW - 2)
    wait_write(_CPW - 1)


def kernel(x, router_w, router_b, gate_w, up_w, down_w):
    xf = x.reshape(T, HIDDEN)
    eid2, wgt2 = _router(xf, router_w, router_b)
    src, wslot, slot = _route(eid2.reshape(-1), wgt2.reshape(-1))
    xb32 = lax.bitcast_convert_type(
        xf.astype(jnp.bfloat16).reshape(T, _HW2, 2), jnp.int32)
    disp32 = _gather_disp(xb32, src)
    disp = lax.bitcast_convert_type(disp32, jnp.bfloat16).reshape(ROWS, HIDDEN)
    outbuf = _ffn(disp, gate_w, up_w, down_w, wslot.reshape(ROWS, 1))
    out = _combine(outbuf, slot)
    return out.reshape(B, S, HIDDEN)


# speedup vs baseline: 1.9210x; 1.9210x over previous
import functools
import math

import jax
import jax.numpy as jnp
from jax import lax
from jax.experimental import pallas as pl
from jax.experimental.pallas import tpu as pltpu
from jax.experimental.pallas import tpu_sc as plsc

B, S, HIDDEN = 1, 2048, 1024
FFN = 2048
E = 8
K = 2
T = B * S
CAP = int(math.ceil(T * K / E * 1.25))
CAPP = 656
DROP_POS = CAPP - 1
ROWS = E * CAPP
NP = 2 * T

_SC_MESH = plsc.VectorSubcoreMesh(
    core_axis_name="c", subcore_axis_name="s", num_cores=2, num_subcores=16)
_NW = 32


def _wid():
    return lax.axis_index("s") * 2 + lax.axis_index("c")


def _router_body(x_ref, rw_ref, rb_ref, eid_ref, wgt_ref):
    xf = x_ref[...]
    logits = lax.dot_general(xf, rw_ref[...], (((1,), (1,)), ((), ())),
                             preferred_element_type=jnp.float32)
    logits = logits + rb_ref[...]
    probs = jax.nn.softmax(logits, axis=-1)
    ei = lax.broadcasted_iota(jnp.int32, (T, E), 1)
    m1 = jnp.max(probs, axis=1, keepdims=True)
    i1 = jnp.min(jnp.where(probs == m1, ei, E), axis=1, keepdims=True)
    pm = jnp.where(ei == i1, -1.0, probs)
    m2 = jnp.max(pm, axis=1, keepdims=True)
    i2 = jnp.min(jnp.where(pm == m2, ei, E), axis=1, keepdims=True)
    s = m1 + m2
    eid_ref[...] = jnp.concatenate([i1, i2], axis=1)
    wgt_ref[...] = jnp.concatenate([m1 / s, m2 / s], axis=1)


def _router(xf, router_w, router_b):
    return pl.pallas_call(
        _router_body,
        out_shape=(jax.ShapeDtypeStruct((T, K), jnp.int32),
                   jax.ShapeDtypeStruct((T, K), jnp.float32)),
    )(xf, router_w, router_b.reshape(1, E))


@functools.partial(
    pl.kernel,
    out_type=(jax.ShapeDtypeStruct((ROWS,), jnp.int32),
              jax.ShapeDtypeStruct((ROWS,), jnp.float32),
              jax.ShapeDtypeStruct((NP,), jnp.int32)),
    mesh=_SC_MESH,
    scratch_types=[pltpu.VMEM((NP,), jnp.int32),
                   pltpu.VMEM((NP,), jnp.float32),
                   pltpu.VMEM((ROWS,), jnp.int32),
                   pltpu.VMEM((ROWS,), jnp.float32),
                   pltpu.VMEM((NP,), jnp.int32),
                   pltpu.VMEM((16,), jnp.int32)],
    compiler_params=pltpu.CompilerParams(needs_layout_passes=False),
)
def _route(eid_hbm, wgt_hbm, src_hbm, wslot_hbm, slot_hbm,
           eid_v, wgt_v, src_v, wslot_v, slot_v, cnt_v):
    @pl.when(_wid() == 0)
    def _():
        pltpu.sync_copy(eid_hbm, eid_v)
        pltpu.sync_copy(wgt_hbm, wgt_v)
        zi = jnp.zeros((16,), jnp.int32)
        zf = jnp.zeros((16,), jnp.float32)
        cnt_v[...] = zi

        def init_body(i, c):
            src_v[pl.ds(i * 16, 16)] = zi
            wslot_v[pl.ds(i * 16, 16)] = zf
            return c
        lax.fori_loop(0, ROWS // 16, init_body, 0)

        lane = lax.iota(jnp.int32, 16)

        def body(i, c):
            ids = eid_v[pl.ds(i * 16, 16)]
            wv = wgt_v[pl.ds(i * 16, 16)]
            base = plsc.load_gather(cnt_v, [ids])
            prefix = zi
            tot = zi
            for e in range(E):
                sel = ids == e
                ind = jnp.where(sel, 1, 0)
                cs = plsc.cumsum(ind)
                prefix = jnp.where(sel, cs - 1, prefix)
                tot = jnp.where(sel, jnp.max(cs), tot)
            pos = base + prefix
            plsc.store_scatter(cnt_v, [ids], base + tot)
            keep = pos < CAP
            slot = ids * CAPP + jnp.where(keep, pos, DROP_POS)
            slot_v[pl.ds(i * 16, 16)] = slot
            tok = lax.shift_right_logical(i * 16 + lane, 1)
            plsc.store_scatter(src_v, [slot], tok, mask=keep)
            plsc.store_scatter(wslot_v, [slot], wv, mask=keep)
            return c
        lax.fori_loop(0, NP // 16, body, 0)
        pltpu.sync_copy(src_v, src_hbm)
        pltpu.sync_copy(wslot_v, wslot_hbm)
        pltpu.sync_copy(slot_v, slot_hbm)


_FBLK = 512
_NF = FFN // _FBLK


def _ffn_body(x_ref, src_ref, gw_ref, uw_ref, dw_ref, w_ref, out_ref,
              xb_s, disp_s):
    e = pl.program_id(0)
    f = pl.program_id(1)

    @pl.when(jnp.logical_and(e == 0, f == 0))
    def _():
        xb_s[...] = x_ref[...].astype(jnp.bfloat16)

    @pl.when(f == 0)
    def _():
        tid = lax.broadcasted_iota(jnp.int32, (CAPP, T), 1)
        oh = (tid == src_ref[...]).astype(jnp.bfloat16)
        disp_s[...] = lax.dot_general(
            oh, xb_s[...], (((1,), (0,)), ((), ())),
            preferred_element_type=jnp.float32).astype(jnp.bfloat16)

    d = disp_s[...]
    g = lax.dot_general(d, gw_ref[0].astype(jnp.bfloat16),
                        (((1,), (1,)), ((), ())),
                        preferred_element_type=jnp.float32)
    u = lax.dot_general(d, uw_ref[0].astype(jnp.bfloat16),
                        (((1,), (1,)), ((), ())),
                        preferred_element_type=jnp.float32)
    h = ((g * jax.nn.sigmoid(g)) * u).astype(jnp.bfloat16)
    p = lax.dot_general(h, dw_ref[0].astype(jnp.bfloat16),
                        (((1,), (1,)), ((), ())),
                        preferred_element_type=jnp.float32)

    @pl.when(f == 0)
    def _():
        out_ref[...] = p

    @pl.when(f != 0)
    def _():
        out_ref[...] = out_ref[...] + p

    @pl.when(f == _NF - 1)
    def _():
        out_ref[...] = out_ref[...] * w_ref[...]


def _ffn(xf, srccol, gate_w, up_w, down_w, wcol):
    return pl.pallas_call(
        _ffn_body,
        grid=(E, _NF),
        in_specs=[
            pl.BlockSpec((T, HIDDEN), lambda e, f: (0, 0)),
            pl.BlockSpec((CAPP, 1), lambda e, f: (e, 0)),
            pl.BlockSpec((1, _FBLK, HIDDEN), lambda e, f: (e, f, 0)),
            pl.BlockSpec((1, _FBLK, HIDDEN), lambda e, f: (e, f, 0)),
            pl.BlockSpec((1, HIDDEN, _FBLK), lambda e, f: (e, 0, f)),
            pl.BlockSpec((CAPP, 1), lambda e, f: (e, 0)),
        ],
        out_specs=pl.BlockSpec((CAPP, HIDDEN), lambda e, f: (e, 0)),
        out_shape=jax.ShapeDtypeStruct((ROWS, HIDDEN), jnp.float32),
        scratch_shapes=[pltpu.VMEM((T, HIDDEN), jnp.bfloat16),
                        pltpu.VMEM((CAPP, HIDDEN), jnp.bfloat16)],
        compiler_params=pltpu.CompilerParams(
            dimension_semantics=("arbitrary", "arbitrary")),
    )(xf, srccol, gate_w, up_w, down_w, wcol)


_TCH = 16
_CPW = T // _TCH // _NW


@functools.partial(
    pl.kernel,
    out_type=jax.ShapeDtypeStruct((T, HIDDEN), jnp.float32),
    mesh=_SC_MESH,
    scratch_types=[pltpu.VMEM((_CPW, 2 * _TCH), jnp.int32),
                   pltpu.VMEM((2, 2 * _TCH, HIDDEN), jnp.float32),
                   pltpu.VMEM((2, _TCH, HIDDEN), jnp.float32),
                   pltpu.SemaphoreType.DMA,
                   pltpu.SemaphoreType.DMA,
                   pltpu.SemaphoreType.DMA,
                   pltpu.SemaphoreType.DMA],
)
def _combine(outbuf_hbm, slot_hbm, out_hbm, idx_v, rows_v, out_v,
             gsem0, gsem1, wsem0, wsem1):
    w = _wid()
    gsem = (gsem0, gsem1)
    wsem = (wsem0, wsem1)

    def tb(c):
        return (w * _CPW + c) * _TCH

    for c in range(_CPW):
        pltpu.sync_copy(slot_hbm.at[pl.ds(tb(c) * 2, 2 * _TCH)], idx_v.at[c])

    def start_gather(c):
        pltpu.async_copy(outbuf_hbm.at[idx_v.at[c]], rows_v.at[c % 2],
                         gsem[c % 2])

    def wait_gather(c):
        pltpu.make_async_copy(outbuf_hbm.at[idx_v.at[c]], rows_v.at[c % 2],
                              gsem[c % 2]).wait()

    def start_write(c):
        pltpu.async_copy(out_v.at[c % 2], out_hbm.at[pl.ds(tb(c), _TCH)],
                         wsem[c % 2])

    def wait_write(c):
        pltpu.make_async_copy(out_v.at[c % 2], out_hbm.at[pl.ds(tb(c), _TCH)],
                              wsem[c % 2]).wait()

    start_gather(0)
    for c in range(_CPW):
        if c + 1 < _CPW:
            if c - 1 >= 0:
                wait_write(c - 1)
            start_gather(c + 1)
        wait_gather(c)

        def tbody(t, carry, c=c):
            for v in range(HIDDEN // 16):
                col = v * 16
                out_v[c % 2, t, pl.ds(col, 16)] = (
                    rows_v[c % 2, 2 * t, pl.ds(col, 16)]
                    + rows_v[c % 2, 2 * t + 1, pl.ds(col, 16)])
            return carry
        lax.fori_loop(0, _TCH, tbody, 0)
        start_write(c)
    wait_write(_CPW - 2)
    wait_write(_CPW - 1)


def kernel(x, router_w, router_b, gate_w, up_w, down_w):
    xf = x.reshape(T, HIDDEN)
    eid2, wgt2 = _router(xf, router_w, router_b)
    src, wslot, slot = _route(eid2.reshape(-1), wgt2.reshape(-1))
    outbuf = _ffn(xf, src.reshape(ROWS, 1), gate_w, up_w, down_w,
                  wslot.reshape(ROWS, 1))
    out = _combine(outbuf, slot)
    return out.reshape(B, S, HIDDEN)


# speedup vs baseline: 1.9518x; 1.0160x over previous
import functools
import math

import jax
import jax.numpy as jnp
from jax import lax
from jax.experimental import pallas as pl
from jax.experimental.pallas import tpu as pltpu
from jax.experimental.pallas import tpu_sc as plsc

B, S, HIDDEN = 1, 2048, 1024
FFN = 2048
E = 8
K = 2
T = B * S
CAP = int(math.ceil(T * K / E * 1.25))
CAPP = 656
DROP_POS = CAPP - 1
ROWS = E * CAPP
NP = 2 * T

_SC_MESH = plsc.VectorSubcoreMesh(
    core_axis_name="c", subcore_axis_name="s", num_cores=2, num_subcores=16)
_NW = 32


def _wid():
    return lax.axis_index("s") * 2 + lax.axis_index("c")


def _router_body(x_ref, rw_ref, rb_ref, eid_ref, wgt_ref):
    xf = x_ref[...]
    logits = lax.dot_general(xf, rw_ref[...], (((1,), (1,)), ((), ())),
                             preferred_element_type=jnp.float32)
    logits = logits + rb_ref[...]
    probs = jax.nn.softmax(logits, axis=-1)
    ei = lax.broadcasted_iota(jnp.int32, (T, E), 1)
    m1 = jnp.max(probs, axis=1, keepdims=True)
    i1 = jnp.min(jnp.where(probs == m1, ei, E), axis=1, keepdims=True)
    pm = jnp.where(ei == i1, -1.0, probs)
    m2 = jnp.max(pm, axis=1, keepdims=True)
    i2 = jnp.min(jnp.where(pm == m2, ei, E), axis=1, keepdims=True)
    s = m1 + m2
    eid_ref[...] = jnp.concatenate([i1, i2], axis=1)
    wgt_ref[...] = jnp.concatenate([m1 / s, m2 / s], axis=1)


def _router(xf, router_w, router_b):
    return pl.pallas_call(
        _router_body,
        out_shape=(jax.ShapeDtypeStruct((T, K), jnp.int32),
                   jax.ShapeDtypeStruct((T, K), jnp.float32)),
    )(xf, router_w, router_b.reshape(1, E))


@functools.partial(
    pl.kernel,
    out_type=(jax.ShapeDtypeStruct((ROWS,), jnp.int32),
              jax.ShapeDtypeStruct((ROWS,), jnp.float32),
              jax.ShapeDtypeStruct((NP,), jnp.int32)),
    mesh=_SC_MESH,
    scratch_types=[pltpu.VMEM((NP,), jnp.int32),
                   pltpu.VMEM((NP,), jnp.float32),
                   pltpu.VMEM((ROWS,), jnp.int32),
                   pltpu.VMEM((ROWS,), jnp.float32),
                   pltpu.VMEM((NP,), jnp.int32),
                   pltpu.VMEM((16,), jnp.int32)],
    compiler_params=pltpu.CompilerParams(needs_layout_passes=False),
)
def _route(eid_hbm, wgt_hbm, src_hbm, wslot_hbm, slot_hbm,
           eid_v, wgt_v, src_v, wslot_v, slot_v, cnt_v):
    @pl.when(_wid() == 0)
    def _():
        pltpu.sync_copy(eid_hbm, eid_v)
        pltpu.sync_copy(wgt_hbm, wgt_v)
        zi = jnp.zeros((16,), jnp.int32)
        zf = jnp.zeros((16,), jnp.float32)
        cnt_v[...] = zi

        def init_body(i, c):
            src_v[pl.ds(i * 16, 16)] = zi
            wslot_v[pl.ds(i * 16, 16)] = zf
            return c
        lax.fori_loop(0, ROWS // 16, init_body, 0)

        lane = lax.iota(jnp.int32, 16)

        def body(i, c):
            ids = eid_v[pl.ds(i * 16, 16)]
            wv = wgt_v[pl.ds(i * 16, 16)]
            base = plsc.load_gather(cnt_v, [ids])
            prefix = zi
            tot = zi
            for e in range(E):
                sel = ids == e
                ind = jnp.where(sel, 1, 0)
                cs = plsc.cumsum(ind)
                prefix = jnp.where(sel, cs - 1, prefix)
                tot = jnp.where(sel, jnp.max(cs), tot)
            pos = base + prefix
            plsc.store_scatter(cnt_v, [ids], base + tot)
            keep = pos < CAP
            slot = ids * CAPP + jnp.where(keep, pos, DROP_POS)
            slot_v[pl.ds(i * 16, 16)] = slot
            tok = lax.shift_right_logical(i * 16 + lane, 1)
            plsc.store_scatter(src_v, [slot], tok, mask=keep)
            plsc.store_scatter(wslot_v, [slot], wv, mask=keep)
            return c
        lax.fori_loop(0, NP // 16, body, 0)
        pltpu.sync_copy(src_v, src_hbm)
        pltpu.sync_copy(wslot_v, wslot_hbm)
        pltpu.sync_copy(slot_v, slot_hbm)


_FBLK = 1024
_NF = FFN // _FBLK


def _ffn_body(x_ref, src_ref, gw_ref, uw_ref, dw_ref, w_ref, out_ref,
              xb_s, disp_s):
    e = pl.program_id(0)
    f = pl.program_id(1)

    @pl.when(jnp.logical_and(e == 0, f == 0))
    def _():
        xb_s[...] = x_ref[...].astype(jnp.bfloat16)

    @pl.when(f == 0)
    def _():
        tid = lax.broadcasted_iota(jnp.int32, (CAPP, T), 1)
        oh = (tid == src_ref[...]).astype(jnp.bfloat16)
        disp_s[...] = lax.dot_general(
            oh, xb_s[...], (((1,), (0,)), ((), ())),
            preferred_element_type=jnp.float32).astype(jnp.bfloat16)

    d = disp_s[...]
    g = lax.dot_general(d, gw_ref[0].astype(jnp.bfloat16),
                        (((1,), (1,)), ((), ())),
                        preferred_element_type=jnp.float32)
    u = lax.dot_general(d, uw_ref[0].astype(jnp.bfloat16),
                        (((1,), (1,)), ((), ())),
                        preferred_element_type=jnp.float32)
    h = ((g * jax.nn.sigmoid(g)) * u).astype(jnp.bfloat16)
    p = lax.dot_general(h, dw_ref[0].astype(jnp.bfloat16),
                        (((1,), (1,)), ((), ())),
                        preferred_element_type=jnp.float32)

    @pl.when(f == 0)
    def _():
        out_ref[...] = p

    @pl.when(f != 0)
    def _():
        out_ref[...] = out_ref[...] + p

    @pl.when(f == _NF - 1)
    def _():
        out_ref[...] = out_ref[...] * w_ref[...]


def _ffn(xf, srccol, gate_w, up_w, down_w, wcol):
    return pl.pallas_call(
        _ffn_body,
        grid=(E, _NF),
        in_specs=[
            pl.BlockSpec((T, HIDDEN), lambda e, f: (0, 0)),
            pl.BlockSpec((CAPP, 1), lambda e, f: (e, 0)),
            pl.BlockSpec((1, _FBLK, HIDDEN), lambda e, f: (e, f, 0)),
            pl.BlockSpec((1, _FBLK, HIDDEN), lambda e, f: (e, f, 0)),
            pl.BlockSpec((1, HIDDEN, _FBLK), lambda e, f: (e, 0, f)),
            pl.BlockSpec((CAPP, 1), lambda e, f: (e, 0)),
        ],
        out_specs=pl.BlockSpec((CAPP, HIDDEN), lambda e, f: (e, 0)),
        out_shape=jax.ShapeDtypeStruct((ROWS, HIDDEN), jnp.float32),
        scratch_shapes=[pltpu.VMEM((T, HIDDEN), jnp.bfloat16),
                        pltpu.VMEM((CAPP, HIDDEN), jnp.bfloat16)],
        compiler_params=pltpu.CompilerParams(
            dimension_semantics=("arbitrary", "arbitrary")),
    )(xf, srccol, gate_w, up_w, down_w, wcol)


_TCH = 16
_CPW = T // _TCH // _NW


@functools.partial(
    pl.kernel,
    out_type=jax.ShapeDtypeStruct((T, HIDDEN), jnp.float32),
    mesh=_SC_MESH,
    scratch_types=[pltpu.VMEM((_CPW, 2 * _TCH), jnp.int32),
                   pltpu.VMEM((2, 2 * _TCH, HIDDEN), jnp.float32),
                   pltpu.VMEM((2, _TCH, HIDDEN), jnp.float32),
                   pltpu.SemaphoreType.DMA,
                   pltpu.SemaphoreType.DMA,
                   pltpu.SemaphoreType.DMA,
                   pltpu.SemaphoreType.DMA],
)
def _combine(outbuf_hbm, slot_hbm, out_hbm, idx_v, rows_v, out_v,
             gsem0, gsem1, wsem0, wsem1):
    w = _wid()
    gsem = (gsem0, gsem1)
    wsem = (wsem0, wsem1)

    def tb(c):
        return (w * _CPW + c) * _TCH

    for c in range(_CPW):
        pltpu.sync_copy(slot_hbm.at[pl.ds(tb(c) * 2, 2 * _TCH)], idx_v.at[c])

    def start_gather(c):
        pltpu.async_copy(outbuf_hbm.at[idx_v.at[c]], rows_v.at[c % 2],
                         gsem[c % 2])

    def wait_gather(c):
        pltpu.make_async_copy(outbuf_hbm.at[idx_v.at[c]], rows_v.at[c % 2],
                              gsem[c % 2]).wait()

    def start_write(c):
        pltpu.async_copy(out_v.at[c % 2], out_hbm.at[pl.ds(tb(c), _TCH)],
                         wsem[c % 2])

    def wait_write(c):
        pltpu.make_async_copy(out_v.at[c % 2], out_hbm.at[pl.ds(tb(c), _TCH)],
                              wsem[c % 2]).wait()

    start_gather(0)
    for c in range(_CPW):
        if c + 1 < _CPW:
            if c - 1 >= 0:
                wait_write(c - 1)
            start_gather(c + 1)
        wait_gather(c)

        def tbody(t, carry, c=c):
            for v in range(HIDDEN // 16):
                col = v * 16
                out_v[c % 2, t, pl.ds(col, 16)] = (
                    rows_v[c % 2, 2 * t, pl.ds(col, 16)]
                    + rows_v[c % 2, 2 * t + 1, pl.ds(col, 16)])
            return carry
        lax.fori_loop(0, _TCH, tbody, 0)
        start_write(c)
    wait_write(_CPW - 2)
    wait_write(_CPW - 1)


def kernel(x, router_w, router_b, gate_w, up_w, down_w):
    xf = x.reshape(T, HIDDEN)
    eid2, wgt2 = _router(xf, router_w, router_b)
    src, wslot, slot = _route(eid2.reshape(-1), wgt2.reshape(-1))
    outbuf = _ffn(xf, src.reshape(ROWS, 1), gate_w, up_w, down_w,
                  wslot.reshape(ROWS, 1))
    out = _combine(outbuf, slot)
    return out.reshape(B, S, HIDDEN)


# speedup vs baseline: 1.9553x; 1.0018x over previous
import functools
import math

import jax
import jax.numpy as jnp
from jax import lax
from jax.experimental import pallas as pl
from jax.experimental.pallas import tpu as pltpu
from jax.experimental.pallas import tpu_sc as plsc

B, S, HIDDEN = 1, 2048, 1024
FFN = 2048
E = 8
K = 2
T = B * S
CAP = int(math.ceil(T * K / E * 1.25))
CAPP = 656
DROP_POS = CAPP - 1
ROWS = E * CAPP
NP = 2 * T

_SC_MESH = plsc.VectorSubcoreMesh(
    core_axis_name="c", subcore_axis_name="s", num_cores=2, num_subcores=16)
_NW = 32


def _wid():
    return lax.axis_index("s") * 2 + lax.axis_index("c")


def _router_body(x_ref, rw_ref, rb_ref, eid_ref, wgt_ref, xb_ref):
    xf = x_ref[...]
    xb_ref[...] = xf.astype(jnp.bfloat16)
    logits = lax.dot_general(xf, rw_ref[...], (((1,), (1,)), ((), ())),
                             preferred_element_type=jnp.float32)
    logits = logits + rb_ref[...]
    probs = jax.nn.softmax(logits, axis=-1)
    ei = lax.broadcasted_iota(jnp.int32, (T, E), 1)
    m1 = jnp.max(probs, axis=1, keepdims=True)
    i1 = jnp.min(jnp.where(probs == m1, ei, E), axis=1, keepdims=True)
    pm = jnp.where(ei == i1, -1.0, probs)
    m2 = jnp.max(pm, axis=1, keepdims=True)
    i2 = jnp.min(jnp.where(pm == m2, ei, E), axis=1, keepdims=True)
    s = m1 + m2
    eid_ref[...] = jnp.concatenate([i1, i2], axis=1)
    wgt_ref[...] = jnp.concatenate([m1 / s, m2 / s], axis=1)


def _router(xf, router_w, router_b):
    return pl.pallas_call(
        _router_body,
        out_shape=(jax.ShapeDtypeStruct((T, K), jnp.int32),
                   jax.ShapeDtypeStruct((T, K), jnp.float32),
                   jax.ShapeDtypeStruct((T, HIDDEN), jnp.bfloat16)),
    )(xf, router_w, router_b.reshape(1, E))


@functools.partial(
    pl.kernel,
    out_type=(jax.ShapeDtypeStruct((ROWS,), jnp.int32),
              jax.ShapeDtypeStruct((ROWS,), jnp.float32),
              jax.ShapeDtypeStruct((NP,), jnp.int32)),
    mesh=_SC_MESH,
    scratch_types=[pltpu.VMEM((NP,), jnp.int32),
                   pltpu.VMEM((NP,), jnp.float32),
                   pltpu.VMEM((ROWS,), jnp.int32),
                   pltpu.VMEM((ROWS,), jnp.float32),
                   pltpu.VMEM((NP,), jnp.int32),
                   pltpu.VMEM((16,), jnp.int32)],
    compiler_params=pltpu.CompilerParams(needs_layout_passes=False),
)
def _route(eid_hbm, wgt_hbm, src_hbm, wslot_hbm, slot_hbm,
           eid_v, wgt_v, src_v, wslot_v, slot_v, cnt_v):
    @pl.when(_wid() == 0)
    def _():
        pltpu.sync_copy(eid_hbm, eid_v)
        pltpu.sync_copy(wgt_hbm, wgt_v)
        zi = jnp.zeros((16,), jnp.int32)
        zf = jnp.zeros((16,), jnp.float32)
        cnt_v[...] = zi

        def init_body(i, c):
            src_v[pl.ds(i * 16, 16)] = zi
            wslot_v[pl.ds(i * 16, 16)] = zf
            return c
        lax.fori_loop(0, ROWS // 16, init_body, 0)

        lane = lax.iota(jnp.int32, 16)

        def body(i, c):
            ids = eid_v[pl.ds(i * 16, 16)]
            wv = wgt_v[pl.ds(i * 16, 16)]
            base = plsc.load_gather(cnt_v, [ids])
            prefix = zi
            tot = zi
            for e in range(E):
                sel = ids == e
                ind = jnp.where(sel, 1, 0)
                cs = plsc.cumsum(ind)
                prefix = jnp.where(sel, cs - 1, prefix)
                tot = jnp.where(sel, jnp.max(cs), tot)
            pos = base + prefix
            plsc.store_scatter(cnt_v, [ids], base + tot)
            keep = pos < CAP
            slot = ids * CAPP + jnp.where(keep, pos, DROP_POS)
            slot_v[pl.ds(i * 16, 16)] = slot
            tok = lax.shift_right_logical(i * 16 + lane, 1)
            plsc.store_scatter(src_v, [slot], tok, mask=keep)
            plsc.store_scatter(wslot_v, [slot], wv, mask=keep)
            return c
        lax.fori_loop(0, NP // 16, body, 0)
        pltpu.sync_copy(src_v, src_hbm)
        pltpu.sync_copy(wslot_v, wslot_hbm)
        pltpu.sync_copy(slot_v, slot_hbm)


_FBLK = 1024
_NF = FFN // _FBLK


def _ffn_body(xb_ref, src_ref, gw_ref, uw_ref, dw_ref, w_ref, out_ref,
              disp_s):
    f = pl.program_id(1)

    @pl.when(f == 0)
    def _():
        tid = lax.broadcasted_iota(jnp.int32, (CAPP, T), 1)
        oh = (tid == src_ref[...]).astype(jnp.bfloat16)
        disp_s[...] = lax.dot_general(
            oh, xb_ref[...], (((1,), (0,)), ((), ())),
            preferred_element_type=jnp.float32).astype(jnp.bfloat16)

    d = disp_s[...]
    g = lax.dot_general(d, gw_ref[0].astype(jnp.bfloat16),
                        (((1,), (1,)), ((), ())),
                        preferred_element_type=jnp.float32)
    u = lax.dot_general(d, uw_ref[0].astype(jnp.bfloat16),
                        (((1,), (1,)), ((), ())),
                        preferred_element_type=jnp.float32)
    h = ((g * jax.nn.sigmoid(g)) * u).astype(jnp.bfloat16)
    p = lax.dot_general(h, dw_ref[0].astype(jnp.bfloat16),
                        (((1,), (1,)), ((), ())),
                        preferred_element_type=jnp.float32)

    @pl.when(f == 0)
    def _():
        out_ref[...] = p

    @pl.when(f != 0)
    def _():
        out_ref[...] = out_ref[...] + p

    @pl.when(f == _NF - 1)
    def _():
        out_ref[...] = out_ref[...] * w_ref[...]


def _ffn(xb, srccol, gate_w, up_w, down_w, wcol):
    return pl.pallas_call(
        _ffn_body,
        grid=(E, _NF),
        in_specs=[
            pl.BlockSpec((T, HIDDEN), lambda e, f: (0, 0)),
            pl.BlockSpec((CAPP, 1), lambda e, f: (e, 0)),
            pl.BlockSpec((1, _FBLK, HIDDEN), lambda e, f: (e, f, 0)),
            pl.BlockSpec((1, _FBLK, HIDDEN), lambda e, f: (e, f, 0)),
            pl.BlockSpec((1, HIDDEN, _FBLK), lambda e, f: (e, 0, f)),
            pl.BlockSpec((CAPP, 1), lambda e, f: (e, 0)),
        ],
        out_specs=pl.BlockSpec((CAPP, HIDDEN), lambda e, f: (e, 0)),
        out_shape=jax.ShapeDtypeStruct((ROWS, HIDDEN), jnp.float32),
        scratch_shapes=[pltpu.VMEM((CAPP, HIDDEN), jnp.bfloat16)],
        compiler_params=pltpu.CompilerParams(
            dimension_semantics=("arbitrary", "arbitrary")),
    )(xb, srccol, gate_w, up_w, down_w, wcol)


_TCH = 16
_CPW = T // _TCH // _NW


@functools.partial(
    pl.kernel,
    out_type=jax.ShapeDtypeStruct((T, HIDDEN), jnp.float32),
    mesh=_SC_MESH,
    scratch_types=[pltpu.VMEM((_CPW, 2 * _TCH), jnp.int32),
                   pltpu.VMEM((2, 2 * _TCH, HIDDEN), jnp.float32),
                   pltpu.VMEM((2, _TCH, HIDDEN), jnp.float32),
                   pltpu.SemaphoreType.DMA,
                   pltpu.SemaphoreType.DMA,
                   pltpu.SemaphoreType.DMA,
                   pltpu.SemaphoreType.DMA],
)
def _combine(outbuf_hbm, slot_hbm, out_hbm, idx_v, rows_v, out_v,
             gsem0, gsem1, wsem0, wsem1):
    w = _wid()
    gsem = (gsem0, gsem1)
    wsem = (wsem0, wsem1)

    def tb(c):
        return (w * _CPW + c) * _TCH

    for c in range(_CPW):
        pltpu.sync_copy(slot_hbm.at[pl.ds(tb(c) * 2, 2 * _TCH)], idx_v.at[c])

    def start_gather(c):
        pltpu.async_copy(outbuf_hbm.at[idx_v.at[c]], rows_v.at[c % 2],
                         gsem[c % 2])

    def wait_gather(c):
        pltpu.make_async_copy(outbuf_hbm.at[idx_v.at[c]], rows_v.at[c % 2],
                              gsem[c % 2]).wait()

    def start_write(c):
        pltpu.async_copy(out_v.at[c % 2], out_hbm.at[pl.ds(tb(c), _TCH)],
                         wsem[c % 2])

    def wait_write(c):
        pltpu.make_async_copy(out_v.at[c % 2], out_hbm.at[pl.ds(tb(c), _TCH)],
                              wsem[c % 2]).wait()

    start_gather(0)
    for c in range(_CPW):
        if c + 1 < _CPW:
            if c - 1 >= 0:
                wait_write(c - 1)
            start_gather(c + 1)
        wait_gather(c)

        def tbody(t, carry, c=c):
            for v in range(HIDDEN // 16):
                col = v * 16
                out_v[c % 2, t, pl.ds(col, 16)] = (
                    rows_v[c % 2, 2 * t, pl.ds(col, 16)]
                    + rows_v[c % 2, 2 * t + 1, pl.ds(col, 16)])
            return carry
        lax.fori_loop(0, _TCH, tbody, 0)
        start_write(c)
    wait_write(_CPW - 2)
    wait_write(_CPW - 1)


def kernel(x, router_w, router_b, gate_w, up_w, down_w):
    xf = x.reshape(T, HIDDEN)
    eid2, wgt2, xb = _router(xf, router_w, router_b)
    src, wslot, slot = _route(eid2.reshape(-1), wgt2.reshape(-1))
    outbuf = _ffn(xb, src.reshape(ROWS, 1), gate_w, up_w, down_w,
                  wslot.reshape(ROWS, 1))
    out = _combine(outbuf, slot)
    return out.reshape(B, S, HIDDEN)
